# speedup vs baseline: 7.3352x; 7.3352x over previous
import functools

import jax
import jax.numpy as jnp
from jax.experimental import pallas as pl
from jax.experimental.pallas import tpu as pltpu

TOTAL_TOK = 16384
D_IN = 256
D_HIDDEN = 512
D_LATENT = 256
N_BATCHES = 16

BLK = 1024
GRID = TOTAL_TOK // BLK


def _body(x_ref, ids_ref, w1_ref, b1_ref, w2_ref, b2_ref, o_ref, acc_ref, cnt_ref):
    i = pl.program_id(0)

    @pl.when(i == 0)
    def _init():
        acc_ref[...] = jnp.zeros_like(acc_ref)
        cnt_ref[...] = jnp.zeros_like(cnt_ref)

    x = x_ref[...]
    h = jnp.dot(x, w1_ref[...], preferred_element_type=jnp.float32) + b1_ref[...]
    h = jnp.maximum(h, 0.0)
    y = jnp.dot(h, w2_ref[...], preferred_element_type=jnp.float32) + b2_ref[...]

    ids = ids_ref[0, 0, :]
    seg = jax.lax.broadcasted_iota(jnp.int32, (N_BATCHES, BLK), 0)
    onehot = (seg == ids[None, :]).astype(jnp.float32)
    acc_ref[...] += jnp.dot(onehot, y, preferred_element_type=jnp.float32)
    cnt_ref[...] += jnp.broadcast_to(
        jnp.sum(onehot, axis=1, keepdims=True), cnt_ref.shape
    )

    @pl.when(i == GRID - 1)
    def _fin():
        o_ref[...] = acc_ref[...] / jnp.maximum(cnt_ref[...], 1.0)


@jax.jit
def kernel(x_flat, batch, W1, b1, W2, b2):
    ids3 = batch.reshape(GRID, 1, BLK)
    b1r = b1.reshape(1, D_HIDDEN)
    b2r = b2.reshape(1, D_LATENT)
    return pl.pallas_call(
        _body,
        grid=(GRID,),
        in_specs=[
            pl.BlockSpec((BLK, D_IN), lambda i: (i, 0)),
            pl.BlockSpec((1, 1, BLK), lambda i: (i, 0, 0)),
            pl.BlockSpec((D_IN, D_HIDDEN), lambda i: (0, 0)),
            pl.BlockSpec((1, D_HIDDEN), lambda i: (0, 0)),
            pl.BlockSpec((D_HIDDEN, D_LATENT), lambda i: (0, 0)),
            pl.BlockSpec((1, D_LATENT), lambda i: (0, 0)),
        ],
        out_specs=pl.BlockSpec((N_BATCHES, D_LATENT), lambda i: (0, 0)),
        out_shape=jax.ShapeDtypeStruct((N_BATCHES, D_LATENT), jnp.float32),
        scratch_shapes=[
            pltpu.VMEM((N_BATCHES, D_LATENT), jnp.float32),
            pltpu.VMEM((N_BATCHES, D_LATENT), jnp.float32),
        ],
    )(x_flat, ids3, W1, b1r, W2, b2r)
